# baseline (speedup 1.0000x reference)
import functools

import jax
import jax.numpy as jnp
from jax import lax
from jax.experimental import pallas as pl
from jax.experimental.pallas import tpu as pltpu
from jax.experimental.pallas import tpu_sc as plsc

VOCAB = 100000
D = 64
B = 16384
SEQ = 200
NT = 20
S1 = 10

_info = plsc.get_sparse_core_info()
_NC = _info.num_cores
_NS = _info.num_subcores
_NW = _NC * _NS
_ROWS_PER_W = B // _NW
_G = 4
_IDXW = 256
_NCHUNK = _ROWS_PER_W // _G


@functools.partial(
    pl.kernel,
    mesh=plsc.VectorSubcoreMesh(core_axis_name="c", subcore_axis_name="s"),
    out_type=jax.ShapeDtypeStruct((B, SEQ, D), jnp.float32),
    compiler_params=pltpu.CompilerParams(use_tc_tiling_on_sc=False),
    scratch_types=[
        pltpu.VMEM((2, _G, _IDXW), jnp.int32),
        pltpu.VMEM((2, _G, SEQ, D), jnp.float32),
        pltpu.VMEM((2 * NT, D), jnp.float32),
        pltpu.SemaphoreType.DMA,
        pltpu.SemaphoreType.DMA,
        pltpu.SemaphoreType.DMA,
        pltpu.SemaphoreType.DMA,
    ],
)
def _prompt_embed(tok_hbm, wte_hbm, le_hbm, out_hbm,
                  idx_v, gbuf, corr_v, sg0, sg1, sw0, sw1):
    wid = lax.axis_index("s") * _NC + lax.axis_index("c")
    base_row = wid * _ROWS_PER_W

    iota = lax.iota(jnp.int32, 16)
    sem_g = (sg0, sg1)
    sem_w = (sw0, sw1)

    pltpu.sync_copy(le_hbm, corr_v.at[pl.ds(0, NT)])
    pltpu.sync_copy(wte_hbm.at[pl.ds(0, NT)], corr_v.at[pl.ds(NT, NT)])
    for row in range(NT):
        for k in range(0, D, 16):
            a = corr_v[row, pl.ds(k, 16)]
            bvec = corr_v[NT + row, pl.ds(k, 16)]
            corr_v[row, pl.ds(k, 16)] = a - bvec

    def gather_descriptors(b):
        idx_b = idx_v.at[b]
        descs = []
        for r in range(_G):
            descs.append((wte_hbm.at[idx_b.at[r, pl.ds(0, 128)]],
                          gbuf.at[b, r, pl.ds(0, 128)]))
            descs.append((wte_hbm.at[idx_b.at[r, pl.ds(128, SEQ - 128)]],
                          gbuf.at[b, r, pl.ds(128, SEQ - 128)]))
        return descs

    def stage(c, b):
        row0 = base_row + c * _G
        idx_b = idx_v.at[b]
        pltpu.sync_copy(tok_hbm.at[pl.ds(row0, _G)],
                        idx_b.at[pl.ds(0, _G), pl.ds(0, SEQ)])
        for r in range(_G):
            g0 = idx_b[r, pl.ds(0, 16)]
            g1 = idx_b[r, pl.ds(16, 16)]
            t21 = g1[5]
            for w, g in ((0, g0), (16, g1)):
                s = w + iota
                in_r = (s >= 1) & (s <= NT + 1)
                cval = jnp.where(s <= S1, s - 1, s - 2)
                new = jnp.where(in_r, jnp.where(s == S1 + 1, t21, cval), g)
                idx_b[r, pl.ds(w, 16)] = new
        @pl.when(c >= 2)
        def _():
            pltpu.make_async_copy(
                gbuf.at[b], out_hbm.at[pl.ds(row0, _G)], sem_w[b]).wait()
        for src, dst in gather_descriptors(b):
            pltpu.async_copy(src, dst, sem_g[b])

    def finish(c, b):
        row0 = base_row + c * _G
        for src, dst in gather_descriptors(b):
            pltpu.make_async_copy(src, dst, sem_g[b]).wait()
        for r in range(_G):
            for j in range(NT):
                p = 1 + j + (j >= S1)
                for k in range(0, D, 16):
                    gbuf[b, r, p, pl.ds(k, 16)] = (
                        gbuf[b, r, p, pl.ds(k, 16)] + corr_v[j, pl.ds(k, 16)])
        pltpu.async_copy(gbuf.at[b], out_hbm.at[pl.ds(row0, _G)], sem_w[b])

    stage(0, 0)

    def pair_body(o, carry):
        c = o * 2
        stage(c + 1, 1)
        finish(c, 0)
        stage(c + 2, 0)
        finish(c + 1, 1)
        return carry

    lax.fori_loop(0, _NCHUNK // 2 - 1, pair_body, 0)
    c = _NCHUNK - 2
    stage(c + 1, 1)
    finish(c, 0)
    finish(c + 1, 1)
    for b in range(2):
        pltpu.make_async_copy(
            gbuf.at[b], out_hbm.at[pl.ds(base_row, _G)], sem_w[b]).wait()


def kernel(tokens, wte_weight, learned_embedding):
    return _prompt_embed(tokens, wte_weight, learned_embedding)


# speedup vs baseline: 1.4445x; 1.4445x over previous
import functools

import jax
import jax.numpy as jnp
from jax import lax
from jax.experimental import pallas as pl
from jax.experimental.pallas import tpu as pltpu
from jax.experimental.pallas import tpu_sc as plsc

VOCAB = 100000
D = 64
B = 16384
SEQ = 200
NT = 20
S1 = 10

_info = plsc.get_sparse_core_info()
_NC = _info.num_cores
_NS = _info.num_subcores
_NW = _NC * _NS
_ROWS_PER_W = B // _NW
_G = 4
_IDXW = 256
_NCHUNK = _ROWS_PER_W // _G


@functools.partial(
    pl.kernel,
    mesh=plsc.VectorSubcoreMesh(core_axis_name="c", subcore_axis_name="s"),
    out_type=jax.ShapeDtypeStruct((B, SEQ, 128), jnp.float32),
    compiler_params=pltpu.CompilerParams(use_tc_tiling_on_sc=False),
    scratch_types=[
        pltpu.VMEM((2, _G, _IDXW), jnp.int32),
        pltpu.VMEM((2, _G, SEQ, D), jnp.float32),
        pltpu.VMEM((2 * NT, D), jnp.float32),
        pltpu.SemaphoreType.DMA,
        pltpu.SemaphoreType.DMA,
        pltpu.SemaphoreType.DMA,
        pltpu.SemaphoreType.DMA,
    ],
)
def _prompt_embed(tok_hbm, wte_hbm, le_hbm, out_hbm,
                  idx_v, gbuf, corr_v, sg0, sg1, sw0, sw1):
    wid = lax.axis_index("s") * _NC + lax.axis_index("c")
    base_row = wid * _ROWS_PER_W

    iota = lax.iota(jnp.int32, 16)
    sem_g = (sg0, sg1)
    sem_w = (sw0, sw1)

    pltpu.sync_copy(le_hbm, corr_v.at[pl.ds(0, NT)])
    pltpu.sync_copy(wte_hbm.at[pl.ds(0, NT)], corr_v.at[pl.ds(NT, NT)])
    for row in range(NT):
        for k in range(0, D, 16):
            a = corr_v[row, pl.ds(k, 16)]
            bvec = corr_v[NT + row, pl.ds(k, 16)]
            corr_v[row, pl.ds(k, 16)] = a - bvec

    def gather_descriptors(b):
        idx_b = idx_v.at[b]
        descs = []
        for r in range(_G):
            descs.append((wte_hbm.at[idx_b.at[r, pl.ds(0, 128)]],
                          gbuf.at[b, r, pl.ds(0, 128)]))
            descs.append((wte_hbm.at[idx_b.at[r, pl.ds(128, SEQ - 128)]],
                          gbuf.at[b, r, pl.ds(128, SEQ - 128)]))
        return descs

    def stage(c, b):
        row0 = base_row + c * _G
        idx_b = idx_v.at[b]
        pltpu.sync_copy(tok_hbm.at[pl.ds(row0, _G)],
                        idx_b.at[pl.ds(0, _G), pl.ds(0, SEQ)])
        for r in range(_G):
            g0 = idx_b[r, pl.ds(0, 16)]
            g1 = idx_b[r, pl.ds(16, 16)]
            t21 = g1[5]
            for w, g in ((0, g0), (16, g1)):
                s = w + iota
                in_r = (s >= 1) & (s <= NT + 1)
                cval = jnp.where(s <= S1, s - 1, s - 2)
                new = jnp.where(in_r, jnp.where(s == S1 + 1, t21, cval), g)
                idx_b[r, pl.ds(w, 16)] = new
        @pl.when(c >= 2)
        def _():
            pltpu.make_async_copy(
                gbuf.at[b],
                out_hbm.at[pl.ds(row0, _G), pl.ds(0, SEQ), pl.ds(0, D)],
                sem_w[b]).wait()
        for src, dst in gather_descriptors(b):
            pltpu.async_copy(src, dst, sem_g[b])

    def finish(c, b):
        row0 = base_row + c * _G
        for src, dst in gather_descriptors(b):
            pltpu.make_async_copy(src, dst, sem_g[b]).wait()
        for r in range(_G):
            for j in range(NT):
                p = 1 + j + (j >= S1)
                for k in range(0, D, 16):
                    gbuf[b, r, p, pl.ds(k, 16)] = (
                        gbuf[b, r, p, pl.ds(k, 16)] + corr_v[j, pl.ds(k, 16)])
        pltpu.async_copy(
            gbuf.at[b],
            out_hbm.at[pl.ds(row0, _G), pl.ds(0, SEQ), pl.ds(0, D)],
            sem_w[b])

    stage(0, 0)

    def pair_body(o, carry):
        c = o * 2
        stage(c + 1, 1)
        finish(c, 0)
        stage(c + 2, 0)
        finish(c + 1, 1)
        return carry

    lax.fori_loop(0, _NCHUNK // 2 - 1, pair_body, 0)
    c = _NCHUNK - 2
    stage(c + 1, 1)
    finish(c, 0)
    finish(c + 1, 1)
    for b in range(2):
        pltpu.make_async_copy(
            gbuf.at[b],
            out_hbm.at[pl.ds(base_row, _G), pl.ds(0, SEQ), pl.ds(0, D)],
            sem_w[b]).wait()


def kernel(tokens, wte_weight, learned_embedding):
    out = _prompt_embed(tokens, wte_weight, learned_embedding)
    return out[:, :, :D]


# speedup vs baseline: 1.4449x; 1.0002x over previous
import functools

import jax
import jax.numpy as jnp
from jax import lax
from jax.experimental import pallas as pl
from jax.experimental.pallas import tpu as pltpu
from jax.experimental.pallas import tpu_sc as plsc

VOCAB = 100000
D = 64
B = 16384
SEQ = 200
NT = 20
S1 = 10

_info = plsc.get_sparse_core_info()
_NC = _info.num_cores
_NS = _info.num_subcores
_NW = _NC * _NS
_ROWS_PER_W = B // _NW
_G = 2
_IDXW = 256
_NCHUNK = _ROWS_PER_W // _G


@functools.partial(
    pl.kernel,
    mesh=plsc.VectorSubcoreMesh(core_axis_name="c", subcore_axis_name="s"),
    out_type=jax.ShapeDtypeStruct((B, SEQ, 128), jnp.float32),
    compiler_params=pltpu.CompilerParams(use_tc_tiling_on_sc=False),
    scratch_types=[
        pltpu.VMEM((4, _G, _IDXW), jnp.int32),
        pltpu.VMEM((2, _G, SEQ, D), jnp.float32),
        pltpu.VMEM((2 * NT, D), jnp.float32),
        pltpu.SemaphoreType.DMA,
        pltpu.SemaphoreType.DMA,
        pltpu.SemaphoreType.DMA,
        pltpu.SemaphoreType.DMA,
        pltpu.SemaphoreType.DMA,
        pltpu.SemaphoreType.DMA,
        pltpu.SemaphoreType.DMA,
        pltpu.SemaphoreType.DMA,
    ],
)
def _prompt_embed(tok_hbm, wte_hbm, le_hbm, out_hbm,
                  idx_v, gbuf, corr_v, sg0, sg1, sw0, sw1,
                  st0, st1, st2, st3):
    wid = lax.axis_index("s") * _NC + lax.axis_index("c")
    base_row = wid * _ROWS_PER_W

    iota = lax.iota(jnp.int32, 16)
    sem_g = (sg0, sg1)
    sem_w = (sw0, sw1)
    sem_t = (st0, st1, st2, st3)

    pltpu.sync_copy(le_hbm, corr_v.at[pl.ds(0, NT)])
    pltpu.sync_copy(wte_hbm.at[pl.ds(0, NT)], corr_v.at[pl.ds(NT, NT)])
    for row in range(NT):
        for k in range(0, D, 16):
            a = corr_v[row, pl.ds(k, 16)]
            bvec = corr_v[NT + row, pl.ds(k, 16)]
            corr_v[row, pl.ds(k, 16)] = a - bvec

    def tok_descriptor(c, ic):
        row0 = base_row + c * _G
        return (tok_hbm.at[pl.ds(row0, _G)],
                idx_v.at[ic].at[pl.ds(0, _G), pl.ds(0, SEQ)], sem_t[ic])

    def prefetch(c, ic):
        src, dst, sem = tok_descriptor(c, ic)
        pltpu.async_copy(src, dst, sem)

    def gather_descriptors(ic, b):
        idx_b = idx_v.at[ic]
        descs = []
        for r in range(_G):
            descs.append((wte_hbm.at[idx_b.at[r, pl.ds(0, 128)]],
                          gbuf.at[b, r, pl.ds(0, 128)]))
            descs.append((wte_hbm.at[idx_b.at[r, pl.ds(128, SEQ - 128)]],
                          gbuf.at[b, r, pl.ds(128, SEQ - 128)]))
        return descs

    def stage(c, ic, b):
        row0 = base_row + c * _G
        idx_b = idx_v.at[ic]
        src, dst, sem = tok_descriptor(c, ic)
        pltpu.make_async_copy(src, dst, sem).wait()
        for r in range(_G):
            g0 = idx_b[r, pl.ds(0, 16)]
            g1 = idx_b[r, pl.ds(16, 16)]
            t21 = g1[5]
            for w, g in ((0, g0), (16, g1)):
                s = w + iota
                in_r = (s >= 1) & (s <= NT + 1)
                cval = jnp.where(s <= S1, s - 1, s - 2)
                new = jnp.where(in_r, jnp.where(s == S1 + 1, t21, cval), g)
                idx_b[r, pl.ds(w, 16)] = new
        @pl.when(c >= 2)
        def _():
            pltpu.make_async_copy(
                gbuf.at[b],
                out_hbm.at[pl.ds(row0, _G), pl.ds(0, SEQ), pl.ds(0, D)],
                sem_w[b]).wait()
        for src, dst in gather_descriptors(ic, b):
            pltpu.async_copy(src, dst, sem_g[b])

    def finish(c, ic, b):
        row0 = base_row + c * _G
        for src, dst in gather_descriptors(ic, b):
            pltpu.make_async_copy(src, dst, sem_g[b]).wait()
        for r in range(_G):
            for j in range(NT):
                p = 1 + j + (j >= S1)
                for k in range(0, D, 16):
                    gbuf[b, r, p, pl.ds(k, 16)] = (
                        gbuf[b, r, p, pl.ds(k, 16)] + corr_v[j, pl.ds(k, 16)])
        pltpu.async_copy(
            gbuf.at[b],
            out_hbm.at[pl.ds(row0, _G), pl.ds(0, SEQ), pl.ds(0, D)],
            sem_w[b])

    for k in range(3):
        prefetch(k, k)
    stage(0, 0, 0)

    def quad_body(o, carry):
        c = o * 4
        for k in range(4):
            prefetch(c + k + 3, (k + 3) % 4)
            stage(c + k + 1, (k + 1) % 4, (k + 1) % 2)
            finish(c + k, k % 4, k % 2)
        return carry

    lax.fori_loop(0, (_NCHUNK - 4) // 4, quad_body, 0)
    c = _NCHUNK - 4
    prefetch(c + 3, 3)
    for k in range(3):
        stage(c + k + 1, (k + 1) % 4, (k + 1) % 2)
        finish(c + k, k % 4, k % 2)
    finish(c + 3, 3, 1)
    for b in range(2):
        pltpu.make_async_copy(
            gbuf.at[b],
            out_hbm.at[pl.ds(base_row, _G), pl.ds(0, SEQ), pl.ds(0, D)],
            sem_w[b]).wait()


def kernel(tokens, wte_weight, learned_embedding):
    out = _prompt_embed(tokens, wte_weight, learned_embedding)
    return out[:, :, :D]
